# baseline (speedup 1.0000x reference)
import jax
import jax.numpy as jnp
from jax import lax
from jax.experimental import pallas as pl
from jax.experimental.pallas import tpu as pltpu
from jax.experimental.pallas import tpu_sc as plsc

NUM_CORES = 2
NUM_SUBCORES = 16
NUM_WORKERS = NUM_CORES * NUM_SUBCORES
LANES = 16
D = 64
ROWS_PER_TILE = 31250
MAX_CHUNK = 625
CHUNK = 128
N_CHUNKS = 50


def _wid():
    return lax.axis_index("s") * NUM_CORES + lax.axis_index("c")


def _max_body(table_hbm, part_hbm, buf0, buf1, acc_v, s0, s1):
    wid = _wid()
    base = wid * ROWS_PER_TILE

    def chunk_start(j, buf, sem):
        pltpu.async_copy(
            table_hbm.at[pl.ds(base + j * MAX_CHUNK, MAX_CHUNK)], buf, sem)

    def chunk_wait(buf, sem):
        pltpu.make_async_copy(
            table_hbm.at[pl.ds(base, MAX_CHUNK)], buf, sem).wait()

    def chunk_reduce(buf, acc):
        def row_body(r, a):
            for c in range(D // LANES):
                a = jnp.maximum(a, jnp.abs(buf[r, pl.ds(c * LANES, LANES)]))
            return a

        return lax.fori_loop(0, MAX_CHUNK, row_body, acc, unroll=4)

    chunk_start(0, buf0, s0)
    chunk_start(1, buf1, s1)
    n_pairs = ROWS_PER_TILE // MAX_CHUNK // 2

    def body(t, acc):
        chunk_wait(buf0, s0)
        acc = chunk_reduce(buf0, acc)

        @pl.when(t < n_pairs - 1)
        def _():
            chunk_start(2 * t + 2, buf0, s0)

        chunk_wait(buf1, s1)
        acc = chunk_reduce(buf1, acc)

        @pl.when(t < n_pairs - 1)
        def _():
            chunk_start(2 * t + 3, buf1, s1)

        return acc

    acc = lax.fori_loop(0, n_pairs, body, jnp.zeros((LANES,), jnp.float32))
    acc_v[...] = acc
    pltpu.sync_copy(acc_v, part_hbm.at[wid])


def _max_partials(weight):
    mesh = plsc.VectorSubcoreMesh(core_axis_name="c", subcore_axis_name="s")
    f = pl.kernel(
        _max_body,
        out_type=jax.ShapeDtypeStruct((NUM_WORKERS, LANES), jnp.float32),
        mesh=mesh,
        scratch_types=[
            pltpu.VMEM((MAX_CHUNK, D), jnp.float32),
            pltpu.VMEM((MAX_CHUNK, D), jnp.float32),
            pltpu.VMEM((LANES,), jnp.float32),
            pltpu.SemaphoreType.DMA,
            pltpu.SemaphoreType.DMA,
        ],
        compiler_params=pltpu.CompilerParams(use_tc_tiling_on_sc=False),
    )
    return f(weight)


def _gather_body(idx_hbm, table_hbm, out_hbm,
                 idx_v, rows0, rows1, out0, out1,
                 g0, g1, o0, o1):
    wid = _wid()
    base = wid * (N_CHUNKS * CHUNK)

    pltpu.sync_copy(idx_hbm.at[wid], idx_v)

    def quantize(rows_v, out_v):
        def row_body(r, carry):
            for c in range(D // LANES):
                out_v[r, pl.ds(c * LANES, LANES)] = (
                    rows_v[r, pl.ds(c * LANES, LANES)])
            return carry

        lax.fori_loop(0, CHUNK, row_body, 0, unroll=4)

    pltpu.async_copy(table_hbm.at[idx_v.at[0]], rows0, g0)
    pltpu.async_copy(table_hbm.at[idx_v.at[1]], rows1, g1)
    n_pairs = N_CHUNKS // 2

    def out_slice(j):
        return out_hbm.at[pl.ds(base + j * CHUNK, CHUNK)]

    def body(t, carry):
        pltpu.make_async_copy(table_hbm.at[idx_v.at[2 * t]], rows0, g0).wait()

        @pl.when(t > 0)
        def _():
            pltpu.make_async_copy(out0, out_slice(0), o0).wait()

        quantize(rows0, out0)
        pltpu.async_copy(out0, out_slice(2 * t), o0)

        @pl.when(t < n_pairs - 1)
        def _():
            pltpu.async_copy(table_hbm.at[idx_v.at[2 * t + 2]], rows0, g0)

        pltpu.make_async_copy(
            table_hbm.at[idx_v.at[2 * t + 1]], rows1, g1).wait()

        @pl.when(t > 0)
        def _():
            pltpu.make_async_copy(out1, out_slice(0), o1).wait()

        quantize(rows1, out1)
        pltpu.async_copy(out1, out_slice(2 * t + 1), o1)

        @pl.when(t < n_pairs - 1)
        def _():
            pltpu.async_copy(table_hbm.at[idx_v.at[2 * t + 3]], rows1, g1)

        return carry

    lax.fori_loop(0, n_pairs, body, 0)
    pltpu.make_async_copy(out0, out_slice(0), o0).wait()
    pltpu.make_async_copy(out1, out_slice(0), o1).wait()


def _gather_raw(idx3, weight):
    total = NUM_WORKERS * N_CHUNKS * CHUNK
    mesh = plsc.VectorSubcoreMesh(core_axis_name="c", subcore_axis_name="s")
    f = pl.kernel(
        _gather_body,
        out_type=jax.ShapeDtypeStruct((total, D), jnp.float32),
        mesh=mesh,
        scratch_types=[
            pltpu.VMEM((N_CHUNKS, CHUNK), jnp.int32),
            pltpu.VMEM((CHUNK, D), jnp.float32),
            pltpu.VMEM((CHUNK, D), jnp.float32),
            pltpu.VMEM((CHUNK, D), jnp.float32),
            pltpu.VMEM((CHUNK, D), jnp.float32),
            pltpu.SemaphoreType.DMA,
            pltpu.SemaphoreType.DMA,
            pltpu.SemaphoreType.DMA,
            pltpu.SemaphoreType.DMA,
        ],
        compiler_params=pltpu.CompilerParams(use_tc_tiling_on_sc=False),
    )
    return f(idx3, weight)


Q_CHUNK = 320


def _quant_body(raw_hbm, maxv_hbm, out_hbm,
                maxv_v, buf0, buf1, ob0, ob1, g0, g1, o0, o1):
    wid = _wid()
    per_tile = raw_hbm.shape[0] // NUM_WORKERS
    base = wid * per_tile

    pltpu.sync_copy(maxv_hbm, maxv_v)
    vmax = maxv_v[...]
    vneg = -vmax
    vthr = vmax * (2.0 ** -24)

    def quantize(buf, ob):
        def row_body(r, carry):
            for c in range(D // LANES):
                w = buf[r, pl.ds(c * LANES, LANES)]
                ob[r, pl.ds(c * LANES, LANES)] = jnp.where(
                    w > vthr, vmax, vneg)
            return carry

        lax.fori_loop(0, Q_CHUNK, row_body, 0, unroll=4)

    def in_slice(j):
        return raw_hbm.at[pl.ds(base + j * Q_CHUNK, Q_CHUNK)]

    def out_slice(j):
        return out_hbm.at[pl.ds(base + j * Q_CHUNK, Q_CHUNK)]

    pltpu.async_copy(in_slice(0), buf0, g0)
    pltpu.async_copy(in_slice(1), buf1, g1)
    n_pairs = per_tile // Q_CHUNK // 2

    def body(t, carry):
        pltpu.make_async_copy(in_slice(0), buf0, g0).wait()

        @pl.when(t > 0)
        def _():
            pltpu.make_async_copy(ob0, out_slice(0), o0).wait()

        quantize(buf0, ob0)
        pltpu.async_copy(ob0, out_slice(2 * t), o0)

        @pl.when(t < n_pairs - 1)
        def _():
            pltpu.async_copy(in_slice(2 * t + 2), buf0, g0)

        pltpu.make_async_copy(in_slice(0), buf1, g1).wait()

        @pl.when(t > 0)
        def _():
            pltpu.make_async_copy(ob1, out_slice(0), o1).wait()

        quantize(buf1, ob1)
        pltpu.async_copy(ob1, out_slice(2 * t + 1), o1)

        @pl.when(t < n_pairs - 1)
        def _():
            pltpu.async_copy(in_slice(2 * t + 3), buf1, g1)

        return carry

    lax.fori_loop(0, n_pairs, body, 0)
    pltpu.make_async_copy(ob0, out_slice(0), o0).wait()
    pltpu.make_async_copy(ob1, out_slice(0), o1).wait()


def _quant_pass(raw, maxvec):
    mesh = plsc.VectorSubcoreMesh(core_axis_name="c", subcore_axis_name="s")
    f = pl.kernel(
        _quant_body,
        out_type=jax.ShapeDtypeStruct(raw.shape, jnp.float32),
        mesh=mesh,
        scratch_types=[
            pltpu.VMEM((LANES,), jnp.float32),
            pltpu.VMEM((Q_CHUNK, D), jnp.float32),
            pltpu.VMEM((Q_CHUNK, D), jnp.float32),
            pltpu.VMEM((Q_CHUNK, D), jnp.float32),
            pltpu.VMEM((Q_CHUNK, D), jnp.float32),
            pltpu.SemaphoreType.DMA,
            pltpu.SemaphoreType.DMA,
            pltpu.SemaphoreType.DMA,
            pltpu.SemaphoreType.DMA,
        ],
        compiler_params=pltpu.CompilerParams(use_tc_tiling_on_sc=False),
    )
    return f(raw, maxvec)


def kernel(input, weight):
    b, s = input.shape
    total = b * s
    assert NUM_WORKERS * N_CHUNKS * CHUNK == total
    idx3 = input.astype(jnp.int32).reshape(NUM_WORKERS, N_CHUNKS, CHUNK)
    partials = _max_partials(weight)
    raw = _gather_raw(idx3, weight)
    maxvec = jnp.broadcast_to(jnp.max(partials), (LANES,))
    out = _quant_pass(raw, maxvec)
    return out.reshape(b, s, D)


# speedup vs baseline: 1.1062x; 1.1062x over previous
import jax
import jax.numpy as jnp
from jax import lax
from jax.experimental import pallas as pl
from jax.experimental.pallas import tpu as pltpu
from jax.experimental.pallas import tpu_sc as plsc

NUM_CORES = 2
NUM_SUBCORES = 16
NUM_WORKERS = NUM_CORES * NUM_SUBCORES
LANES = 16
D = 64
ROWS_PER_TILE = 31250
MAX_CHUNK = 625
CHUNK = 128
N_CHUNKS = 50


def _wid():
    return lax.axis_index("s") * NUM_CORES + lax.axis_index("c")


def _max_body(table_hbm, part_hbm, buf0, buf1, acc_v, s0, s1):
    wid = _wid()
    base = wid * ROWS_PER_TILE

    def chunk_start(j, buf, sem):
        pltpu.async_copy(
            table_hbm.at[pl.ds(base + j * MAX_CHUNK, MAX_CHUNK)], buf, sem)

    def chunk_wait(buf, sem):
        pltpu.make_async_copy(
            table_hbm.at[pl.ds(base, MAX_CHUNK)], buf, sem).wait()

    def chunk_reduce(buf, acc):
        def row_body(r, a):
            for c in range(D // LANES):
                a = jnp.maximum(a, jnp.abs(buf[r, pl.ds(c * LANES, LANES)]))
            return a

        return lax.fori_loop(0, MAX_CHUNK, row_body, acc, unroll=4)

    chunk_start(0, buf0, s0)
    chunk_start(1, buf1, s1)
    n_pairs = ROWS_PER_TILE // MAX_CHUNK // 2

    def body(t, acc):
        chunk_wait(buf0, s0)
        acc = chunk_reduce(buf0, acc)

        @pl.when(t < n_pairs - 1)
        def _():
            chunk_start(2 * t + 2, buf0, s0)

        chunk_wait(buf1, s1)
        acc = chunk_reduce(buf1, acc)

        @pl.when(t < n_pairs - 1)
        def _():
            chunk_start(2 * t + 3, buf1, s1)

        return acc

    acc = lax.fori_loop(0, n_pairs, body, jnp.zeros((LANES,), jnp.float32))
    acc_v[...] = acc
    pltpu.sync_copy(acc_v, part_hbm.at[wid])


def _max_partials(weight):
    mesh = plsc.VectorSubcoreMesh(core_axis_name="c", subcore_axis_name="s")
    f = pl.kernel(
        _max_body,
        out_type=jax.ShapeDtypeStruct((NUM_WORKERS, LANES), jnp.float32),
        mesh=mesh,
        scratch_types=[
            pltpu.VMEM((MAX_CHUNK, D), jnp.float32),
            pltpu.VMEM((MAX_CHUNK, D), jnp.float32),
            pltpu.VMEM((LANES,), jnp.float32),
            pltpu.SemaphoreType.DMA,
            pltpu.SemaphoreType.DMA,
        ],
        compiler_params=pltpu.CompilerParams(use_tc_tiling_on_sc=False),
    )
    return f(weight)


def _gather_body(idx_hbm, table_hbm, maxv_hbm, out_hbm,
                 idx_v, maxv_v, rows0, rows1, out0, out1,
                 g0, g1, o0, o1):
    wid = _wid()
    base = wid * (N_CHUNKS * CHUNK)

    pltpu.sync_copy(idx_hbm.at[wid], idx_v)
    pltpu.sync_copy(maxv_hbm, maxv_v)

    vmax = maxv_v[...]
    vneg = -vmax
    vthr = vmax * (2.0 ** -24)

    def quantize(rows_v, out_v):
        def row_body(r, carry):
            for c in range(D // LANES):
                w = rows_v[r, pl.ds(c * LANES, LANES)]
                out_v[r, pl.ds(c * LANES, LANES)] = jnp.where(
                    w > vthr, vmax, vneg)
            return carry

        lax.fori_loop(0, CHUNK, row_body, 0, unroll=4)

    pltpu.async_copy(table_hbm.at[idx_v.at[0]], rows0, g0)
    pltpu.async_copy(table_hbm.at[idx_v.at[1]], rows1, g1)
    n_pairs = N_CHUNKS // 2

    def out_slice(j):
        return out_hbm.at[pl.ds(base + j * CHUNK, CHUNK)]

    def body(t, carry):
        pltpu.make_async_copy(table_hbm.at[idx_v.at[2 * t]], rows0, g0).wait()

        @pl.when(t > 0)
        def _():
            pltpu.make_async_copy(out0, out_slice(0), o0).wait()

        quantize(rows0, out0)
        pltpu.async_copy(out0, out_slice(2 * t), o0)

        @pl.when(t < n_pairs - 1)
        def _():
            pltpu.async_copy(table_hbm.at[idx_v.at[2 * t + 2]], rows0, g0)

        pltpu.make_async_copy(
            table_hbm.at[idx_v.at[2 * t + 1]], rows1, g1).wait()

        @pl.when(t > 0)
        def _():
            pltpu.make_async_copy(out1, out_slice(0), o1).wait()

        quantize(rows1, out1)
        pltpu.async_copy(out1, out_slice(2 * t + 1), o1)

        @pl.when(t < n_pairs - 1)
        def _():
            pltpu.async_copy(table_hbm.at[idx_v.at[2 * t + 3]], rows1, g1)

        return carry

    lax.fori_loop(0, n_pairs, body, 0)
    pltpu.make_async_copy(out0, out_slice(0), o0).wait()
    pltpu.make_async_copy(out1, out_slice(0), o1).wait()


def _gather_quant(idx3, weight, maxvec):
    total = NUM_WORKERS * N_CHUNKS * CHUNK
    mesh = plsc.VectorSubcoreMesh(core_axis_name="c", subcore_axis_name="s")
    f = pl.kernel(
        _gather_body,
        out_type=jax.ShapeDtypeStruct((total, D), jnp.float32),
        mesh=mesh,
        scratch_types=[
            pltpu.VMEM((N_CHUNKS, CHUNK), jnp.int32),
            pltpu.VMEM((LANES,), jnp.float32),
            pltpu.VMEM((CHUNK, D), jnp.float32),
            pltpu.VMEM((CHUNK, D), jnp.float32),
            pltpu.VMEM((CHUNK, D), jnp.float32),
            pltpu.VMEM((CHUNK, D), jnp.float32),
            pltpu.SemaphoreType.DMA,
            pltpu.SemaphoreType.DMA,
            pltpu.SemaphoreType.DMA,
            pltpu.SemaphoreType.DMA,
        ],
        compiler_params=pltpu.CompilerParams(use_tc_tiling_on_sc=False),
    )
    return f(idx3, weight, maxvec)


def kernel(input, weight):
    b, s = input.shape
    total = b * s
    assert NUM_WORKERS * N_CHUNKS * CHUNK == total
    idx3 = input.astype(jnp.int32).reshape(NUM_WORKERS, N_CHUNKS, CHUNK)
    partials = _max_partials(weight)
    maxvec = jnp.broadcast_to(jnp.max(partials), (LANES,))
    out = _gather_quant(idx3, weight, maxvec)
    return out.reshape(b, s, D)
